# Initial kernel scaffold; baseline (speedup 1.0000x reference)
import functools

import jax
import jax.numpy as jnp
from jax import lax
from jax.experimental import pallas as pl
from jax.experimental.pallas import tpu as pltpu
from jax.experimental.pallas import tpu_sc as plsc

N_NODES = 10000
NPAD = 10240
D = 128
NC = 2
NS = 16
NW = NC * NS
K = 128
ROWBLK = 1024
F32 = jnp.float32
I32 = jnp.int32


def _sc_mesh():
    return plsc.VectorSubcoreMesh(core_axis_name="c", subcore_axis_name="s")


def _deg_body(epw, col_hbm, ew_hbm, degp_hbm, colb, ewb, degv):
    cid = lax.axis_index("c")
    sid = lax.axis_index("s")
    wid = sid * NC + cid

    def zero(i, c):
        degv[pl.ds(i * 16, 16)] = jnp.zeros((16,), F32)
        return c

    lax.fori_loop(0, NPAD // 16, zero, 0)
    pltpu.sync_copy(col_hbm.at[pl.ds(wid * epw, epw)], colb)
    pltpu.sync_copy(ew_hbm.at[pl.ds(wid * epw, epw)], ewb)

    def edge(i, c):
        b = i * 16
        plsc.addupdate_scatter(degv, [colb[pl.ds(b, 16)]], ewb[pl.ds(b, 16)])
        return c

    lax.fori_loop(0, epw // 16, edge, 0)
    pltpu.sync_copy(degv, degp_hbm.at[wid])


def _sc_degree(col_p, ew_p, epw):
    kfn = functools.partial(
        pl.kernel,
        out_type=jax.ShapeDtypeStruct((NW, NPAD), F32),
        mesh=_sc_mesh(),
        scratch_types=[
            pltpu.VMEM((epw,), I32),
            pltpu.VMEM((epw,), F32),
            pltpu.VMEM((NPAD,), F32),
        ],
    )(functools.partial(_deg_body, epw))
    return kfn(col_p, ew_p)


def _msg_body(nchunks, epw, zs_hbm, row_hbm, col_hbm, ew_hbm, out_hbm,
              acc_sh, rowv, colv, ewv, rows, sem):
    cid = lax.axis_index("c")
    sid = lax.axis_index("s")
    wid = sid * NC + cid
    rows_per_tile = NPAD // NS

    def zrow(i, c):
        for g in range(D // 16):
            rows[i, pl.ds(g * 16, 16)] = jnp.zeros((16,), F32)
        return c

    lax.fori_loop(0, K, zrow, 0)

    def zacc(j, c):
        pltpu.sync_copy(rows, acc_sh.at[pl.ds(sid * rows_per_tile + j * K, K)])
        return c

    lax.fori_loop(0, rows_per_tile // K, zacc, 0)
    plsc.subcore_barrier()

    base0 = wid * epw

    def chunk(j, c):
        b = base0 + j * K
        pltpu.sync_copy(row_hbm.at[pl.ds(b, K)], rowv)
        pltpu.sync_copy(col_hbm.at[pl.ds(b, K)], colv)
        pltpu.sync_copy(ew_hbm.at[pl.ds(b, K)], ewv)
        pltpu.async_copy(zs_hbm.at[rowv], rows, sem).wait()

        def scale(k, c2):
            spl = plsc.load_gather(ewv, [jnp.full((16,), k, I32)])
            for g in range(D // 16):
                seg = rows[k, pl.ds(g * 16, 16)]
                rows[k, pl.ds(g * 16, 16)] = seg * spl
            return c2

        lax.fori_loop(0, K, scale, 0)
        pltpu.sync_copy(rows, acc_sh.at[colv], add=True)
        return c

    lax.fori_loop(0, nchunks, chunk, 0)
    plsc.subcore_barrier()

    def wout(j, c):
        r = sid * rows_per_tile + j * K
        pltpu.sync_copy(acc_sh.at[pl.ds(r, K)], rows)
        pltpu.sync_copy(rows, out_hbm.at[cid, pl.ds(r, K)])
        return c

    lax.fori_loop(0, rows_per_tile // K, wout, 0)


def _sc_message(zs, row_p, col_p, ew_p, epw):
    nchunks = epw // K
    kfn = functools.partial(
        pl.kernel,
        out_type=jax.ShapeDtypeStruct((NC, NPAD, D), F32),
        mesh=_sc_mesh(),
        scratch_types=[
            pltpu.VMEM_SHARED((NPAD, D), F32),
            pltpu.VMEM((K,), I32),
            pltpu.VMEM((K,), I32),
            pltpu.VMEM((K,), F32),
            pltpu.VMEM((K, D), F32),
            pltpu.SemaphoreType.DMA,
        ],
    )(functools.partial(_msg_body, nchunks, epw))
    return kfn(zs, row_p, col_p, ew_p)


def _mm_body(x_ref, w_ref, o_ref):
    o_ref[...] = jnp.dot(x_ref[...], w_ref[...], preferred_element_type=F32)


def _tc_matmul(x, w):
    return pl.pallas_call(
        _mm_body,
        grid=(NPAD // ROWBLK,),
        in_specs=[
            pl.BlockSpec((ROWBLK, D), lambda i: (i, 0)),
            pl.BlockSpec((D, D), lambda i: (0, 0)),
        ],
        out_specs=pl.BlockSpec((ROWBLK, D), lambda i: (i, 0)),
        out_shape=jax.ShapeDtypeStruct((NPAD, D), F32),
    )(x, w)


def _scale1_body(z_ref, deg_ref, zs_ref, dis_ref):
    dis = lax.rsqrt(deg_ref[...] + 1.0)
    dis_ref[...] = dis
    zs_ref[...] = z_ref[...] * dis


def _tc_scale1(z, deg_col):
    return pl.pallas_call(
        _scale1_body,
        grid=(NPAD // ROWBLK,),
        in_specs=[
            pl.BlockSpec((ROWBLK, D), lambda i: (i, 0)),
            pl.BlockSpec((ROWBLK, 1), lambda i: (i, 0)),
        ],
        out_specs=[
            pl.BlockSpec((ROWBLK, D), lambda i: (i, 0)),
            pl.BlockSpec((ROWBLK, 1), lambda i: (i, 0)),
        ],
        out_shape=[
            jax.ShapeDtypeStruct((NPAD, D), F32),
            jax.ShapeDtypeStruct((NPAD, 1), F32),
        ],
    )(z, deg_col)


def _mid_body(acc_ref, zs_ref, dis_ref, w_ref, b_ref, o_ref):
    dis = dis_ref[...]
    pre = (acc_ref[0] + acc_ref[1] + zs_ref[...]) * dis + b_ref[...]
    h = jnp.maximum(pre, 0.0)
    o_ref[...] = jnp.dot(h, w_ref[...], preferred_element_type=F32) * dis


def _tc_mid(acc, zs, dis_col, w2, b1):
    return pl.pallas_call(
        _mid_body,
        grid=(NPAD // ROWBLK,),
        in_specs=[
            pl.BlockSpec((NC, ROWBLK, D), lambda i: (0, i, 0)),
            pl.BlockSpec((ROWBLK, D), lambda i: (i, 0)),
            pl.BlockSpec((ROWBLK, 1), lambda i: (i, 0)),
            pl.BlockSpec((D, D), lambda i: (0, 0)),
            pl.BlockSpec((1, D), lambda i: (0, 0)),
        ],
        out_specs=pl.BlockSpec((ROWBLK, D), lambda i: (i, 0)),
        out_shape=jax.ShapeDtypeStruct((NPAD, D), F32),
    )(acc, zs, dis_col, w2, b1)


def _final_body(acc_ref, zs_ref, dis_ref, b_ref, o_ref):
    o_ref[...] = ((acc_ref[0] + acc_ref[1] + zs_ref[...]) * dis_ref[...]
                  + b_ref[...])


def _tc_final(acc, zs, dis_col, b2):
    return pl.pallas_call(
        _final_body,
        grid=(NPAD // ROWBLK,),
        in_specs=[
            pl.BlockSpec((NC, ROWBLK, D), lambda i: (0, i, 0)),
            pl.BlockSpec((ROWBLK, D), lambda i: (i, 0)),
            pl.BlockSpec((ROWBLK, 1), lambda i: (i, 0)),
            pl.BlockSpec((1, D), lambda i: (0, 0)),
        ],
        out_specs=pl.BlockSpec((ROWBLK, D), lambda i: (i, 0)),
        out_shape=jax.ShapeDtypeStruct((NPAD, D), F32),
    )(acc, zs, dis_col, b2)


def kernel(x, edge_index, weight, W1, b1, W2, b2):
    e = weight.shape[0]
    epw = ((e + NW * K - 1) // (NW * K)) * K
    e_pad = epw * NW

    row = edge_index[0].astype(I32)
    col = edge_index[1].astype(I32)
    row_p = jnp.pad(row, (0, e_pad - e))
    col_p = jnp.pad(col, (0, e_pad - e))
    ew_p = jnp.pad(weight.astype(F32), (0, e_pad - e))
    x_pad = jnp.pad(x, ((0, NPAD - x.shape[0]), (0, 0)))

    z1 = _tc_matmul(x_pad, W1)
    degp = _sc_degree(col_p, ew_p, epw)
    deg_col = jnp.sum(degp, axis=0).reshape(NPAD, 1)
    zs1, dis_col = _tc_scale1(z1, deg_col)
    acc1 = _sc_message(zs1, row_p, col_p, ew_p, epw)
    zs2 = _tc_mid(acc1, zs1, dis_col, W2, b1.reshape(1, D))
    acc2 = _sc_message(zs2, row_p, col_p, ew_p, epw)
    out = _tc_final(acc2, zs2, dis_col, b2.reshape(1, D))
    return out[:N_NODES]


# speedup vs baseline: 9.2645x; 9.2645x over previous
import functools

import jax
import jax.numpy as jnp
from jax import lax
from jax.experimental import pallas as pl
from jax.experimental.pallas import tpu as pltpu
from jax.experimental.pallas import tpu_sc as plsc

N_NODES = 10000
NPAD = 10240
D = 128
NC = 2
NS = 16
NW = NC * NS
K = 128
ROWBLK = 1024
F32 = jnp.float32
I32 = jnp.int32


def _sc_mesh():
    return plsc.VectorSubcoreMesh(core_axis_name="c", subcore_axis_name="s")


def _deg_body(epw, col_hbm, ew_hbm, degp_hbm, colb, ewb, degv):
    cid = lax.axis_index("c")
    sid = lax.axis_index("s")
    wid = sid * NC + cid

    def zero(i, c):
        degv[pl.ds(i * 16, 16)] = jnp.zeros((16,), F32)
        return c

    lax.fori_loop(0, NPAD // 16, zero, 0)
    pltpu.sync_copy(col_hbm.at[pl.ds(wid * epw, epw)], colb)
    pltpu.sync_copy(ew_hbm.at[pl.ds(wid * epw, epw)], ewb)

    def edge(i, c):
        b = i * 16
        plsc.addupdate_scatter(degv, [colb[pl.ds(b, 16)]], ewb[pl.ds(b, 16)])
        return c

    lax.fori_loop(0, epw // 16, edge, 0)
    pltpu.sync_copy(degv, degp_hbm.at[wid])


def _sc_degree(col_p, ew_p, epw):
    kfn = functools.partial(
        pl.kernel,
        out_type=jax.ShapeDtypeStruct((NW, NPAD), F32),
        mesh=_sc_mesh(),
        compiler_params=pltpu.CompilerParams(needs_layout_passes=False),
        scratch_types=[
            pltpu.VMEM((epw,), I32),
            pltpu.VMEM((epw,), F32),
            pltpu.VMEM((NPAD,), F32),
        ],
    )(functools.partial(_deg_body, epw))
    return kfn(col_p, ew_p)


def _msg_body(nchunks, epw, zs_hbm, row_hbm, col_hbm, ew_hbm, out_hbm,
              acc_sh, rowv, colv, ewv, rows, sem):
    cid = lax.axis_index("c")
    sid = lax.axis_index("s")
    wid = sid * NC + cid
    rows_per_tile = NPAD // NS

    def zrow(i, c):
        for g in range(D // 16):
            rows[i, pl.ds(g * 16, 16)] = jnp.zeros((16,), F32)
        return c

    lax.fori_loop(0, K, zrow, 0)

    def zacc(j, c):
        pltpu.sync_copy(rows, acc_sh.at[pl.ds(sid * rows_per_tile + j * K, K)])
        return c

    lax.fori_loop(0, rows_per_tile // K, zacc, 0)
    plsc.subcore_barrier()

    base0 = wid * epw

    def chunk(j, c):
        b = base0 + j * K
        pltpu.sync_copy(row_hbm.at[pl.ds(b, K)], rowv)
        pltpu.sync_copy(col_hbm.at[pl.ds(b, K)], colv)
        pltpu.sync_copy(ew_hbm.at[pl.ds(b, K)], ewv)
        pltpu.async_copy(zs_hbm.at[rowv], rows, sem).wait()

        def scale(k, c2):
            spl = plsc.load_gather(ewv, [jnp.full((16,), k, I32)])
            for g in range(D // 16):
                seg = rows[k, pl.ds(g * 16, 16)]
                rows[k, pl.ds(g * 16, 16)] = seg * spl
            return c2

        lax.fori_loop(0, K, scale, 0)
        pltpu.sync_copy(rows, acc_sh.at[colv], add=True)
        return c

    lax.fori_loop(0, nchunks, chunk, 0)
    plsc.subcore_barrier()

    def wout(j, c):
        r = sid * rows_per_tile + j * K
        pltpu.sync_copy(acc_sh.at[pl.ds(r, K)], rows)
        pltpu.sync_copy(rows, out_hbm.at[cid, pl.ds(r, K)])
        return c

    lax.fori_loop(0, rows_per_tile // K, wout, 0)


def _sc_message(zs, row_p, col_p, ew_p, epw):
    nchunks = epw // K
    kfn = functools.partial(
        pl.kernel,
        out_type=jax.ShapeDtypeStruct((NC, NPAD, D), F32),
        mesh=_sc_mesh(),
        compiler_params=pltpu.CompilerParams(needs_layout_passes=False),
        scratch_types=[
            pltpu.VMEM_SHARED((NPAD, D), F32),
            pltpu.VMEM((K,), I32),
            pltpu.VMEM((K,), I32),
            pltpu.VMEM((K,), F32),
            pltpu.VMEM((K, D), F32),
            pltpu.SemaphoreType.DMA,
        ],
    )(functools.partial(_msg_body, nchunks, epw))
    return kfn(zs, row_p, col_p, ew_p)


def _mm_body(x_ref, w_ref, o_ref):
    o_ref[...] = jnp.dot(x_ref[...], w_ref[...], preferred_element_type=F32)


def _tc_matmul(x, w):
    return pl.pallas_call(
        _mm_body,
        grid=(NPAD // ROWBLK,),
        in_specs=[
            pl.BlockSpec((ROWBLK, D), lambda i: (i, 0)),
            pl.BlockSpec((D, D), lambda i: (0, 0)),
        ],
        out_specs=pl.BlockSpec((ROWBLK, D), lambda i: (i, 0)),
        out_shape=jax.ShapeDtypeStruct((NPAD, D), F32),
    )(x, w)


def _scale1_body(z_ref, deg_ref, zs_ref, dis_ref):
    dis = lax.rsqrt(deg_ref[...] + 1.0)
    dis_ref[...] = dis
    zs_ref[...] = z_ref[...] * dis


def _tc_scale1(z, deg_col):
    return pl.pallas_call(
        _scale1_body,
        grid=(NPAD // ROWBLK,),
        in_specs=[
            pl.BlockSpec((ROWBLK, D), lambda i: (i, 0)),
            pl.BlockSpec((ROWBLK, 1), lambda i: (i, 0)),
        ],
        out_specs=[
            pl.BlockSpec((ROWBLK, D), lambda i: (i, 0)),
            pl.BlockSpec((ROWBLK, 1), lambda i: (i, 0)),
        ],
        out_shape=[
            jax.ShapeDtypeStruct((NPAD, D), F32),
            jax.ShapeDtypeStruct((NPAD, 1), F32),
        ],
    )(z, deg_col)


def _mid_body(acc_ref, zs_ref, dis_ref, w_ref, b_ref, o_ref):
    dis = dis_ref[...]
    pre = (acc_ref[0] + acc_ref[1] + zs_ref[...]) * dis + b_ref[...]
    h = jnp.maximum(pre, 0.0)
    o_ref[...] = jnp.dot(h, w_ref[...], preferred_element_type=F32) * dis


def _tc_mid(acc, zs, dis_col, w2, b1):
    return pl.pallas_call(
        _mid_body,
        grid=(NPAD // ROWBLK,),
        in_specs=[
            pl.BlockSpec((NC, ROWBLK, D), lambda i: (0, i, 0)),
            pl.BlockSpec((ROWBLK, D), lambda i: (i, 0)),
            pl.BlockSpec((ROWBLK, 1), lambda i: (i, 0)),
            pl.BlockSpec((D, D), lambda i: (0, 0)),
            pl.BlockSpec((1, D), lambda i: (0, 0)),
        ],
        out_specs=pl.BlockSpec((ROWBLK, D), lambda i: (i, 0)),
        out_shape=jax.ShapeDtypeStruct((NPAD, D), F32),
    )(acc, zs, dis_col, w2, b1)


def _final_body(acc_ref, zs_ref, dis_ref, b_ref, o_ref):
    o_ref[...] = ((acc_ref[0] + acc_ref[1] + zs_ref[...]) * dis_ref[...]
                  + b_ref[...])


def _tc_final(acc, zs, dis_col, b2):
    return pl.pallas_call(
        _final_body,
        grid=(NPAD // ROWBLK,),
        in_specs=[
            pl.BlockSpec((NC, ROWBLK, D), lambda i: (0, i, 0)),
            pl.BlockSpec((ROWBLK, D), lambda i: (i, 0)),
            pl.BlockSpec((ROWBLK, 1), lambda i: (i, 0)),
            pl.BlockSpec((1, D), lambda i: (0, 0)),
        ],
        out_specs=pl.BlockSpec((ROWBLK, D), lambda i: (i, 0)),
        out_shape=jax.ShapeDtypeStruct((NPAD, D), F32),
    )(acc, zs, dis_col, b2)


def kernel(x, edge_index, weight, W1, b1, W2, b2):
    e = weight.shape[0]
    epw = ((e + NW * K - 1) // (NW * K)) * K
    e_pad = epw * NW

    row = edge_index[0].astype(I32)
    col = edge_index[1].astype(I32)
    row_p = jnp.pad(row, (0, e_pad - e))
    col_p = jnp.pad(col, (0, e_pad - e))
    ew_p = jnp.pad(weight.astype(F32), (0, e_pad - e))
    x_pad = jnp.pad(x, ((0, NPAD - x.shape[0]), (0, 0)))

    z1 = _tc_matmul(x_pad, W1)
    degp = _sc_degree(col_p, ew_p, epw)
    deg_col = jnp.sum(degp, axis=0).reshape(NPAD, 1)
    zs1, dis_col = _tc_scale1(z1, deg_col)
    acc1 = _sc_message(zs1, row_p, col_p, ew_p, epw)
    zs2 = _tc_mid(acc1, zs1, dis_col, W2, b1.reshape(1, D))
    acc2 = _sc_message(zs2, row_p, col_p, ew_p, epw)
    out = _tc_final(acc2, zs2, dis_col, b2.reshape(1, D))
    return out[:N_NODES]


# speedup vs baseline: 15.3196x; 1.6536x over previous
import functools

import jax
import jax.numpy as jnp
from jax import lax
from jax.experimental import pallas as pl
from jax.experimental.pallas import tpu as pltpu
from jax.experimental.pallas import tpu_sc as plsc

N_NODES = 10000
NPAD = 10240
D = 128
NC = 2
NS = 16
NW = NC * NS
K = 128
ROWBLK = 1024
F32 = jnp.float32
I32 = jnp.int32


def _sc_mesh():
    return plsc.VectorSubcoreMesh(core_axis_name="c", subcore_axis_name="s")


def _deg_body(epw, col_hbm, ew_hbm, degp_hbm, colb, ewb, degv):
    cid = lax.axis_index("c")
    sid = lax.axis_index("s")
    wid = sid * NC + cid

    def zero(i, c):
        degv[pl.ds(i * 16, 16)] = jnp.zeros((16,), F32)
        return c

    lax.fori_loop(0, NPAD // 16, zero, 0)
    pltpu.sync_copy(col_hbm.at[pl.ds(wid * epw, epw)], colb)
    pltpu.sync_copy(ew_hbm.at[pl.ds(wid * epw, epw)], ewb)

    def edge(i, c):
        b = i * 16
        plsc.addupdate_scatter(degv, [colb[pl.ds(b, 16)]], ewb[pl.ds(b, 16)])
        return c

    lax.fori_loop(0, epw // 16, edge, 0)
    pltpu.sync_copy(degv, degp_hbm.at[wid])


def _sc_degree(col_p, ew_p, epw):
    kfn = functools.partial(
        pl.kernel,
        out_type=jax.ShapeDtypeStruct((NW, NPAD), F32),
        mesh=_sc_mesh(),
        compiler_params=pltpu.CompilerParams(needs_layout_passes=False),
        scratch_types=[
            pltpu.VMEM((epw,), I32),
            pltpu.VMEM((epw,), F32),
            pltpu.VMEM((NPAD,), F32),
        ],
    )(functools.partial(_deg_body, epw))
    return kfn(col_p, ew_p)


NBUF = 3
DH = D // NC


def _msg_body(nchunks, unroll, zs_hbm, row_hbm, col_hbm, ew_hbm, out_hbm,
              acc_sh, rowb, colb, ewb, rows0, rows1, rows2,
              sg0, sg1, sg2, ss0, ss1, ss2):
    cid = lax.axis_index("c")
    sid = lax.axis_index("s")
    rows_per_tile = NPAD // NS
    rows = [rows0, rows1, rows2]
    sg = [sg0, sg1, sg2]
    ss = [ss0, ss1, ss2]
    zs_half = zs_hbm.at[cid]

    pltpu.sync_copy(row_hbm.at[sid], rowb)
    pltpu.sync_copy(col_hbm.at[sid], colb)
    pltpu.sync_copy(ew_hbm.at[sid], ewb)

    @plsc.parallel_loop(0, K)
    def _(i):
        for g in range(DH // 16):
            rows0[i, pl.ds(g * 16, 16)] = jnp.zeros((16,), F32)

    def zacc(j, c):
        pltpu.sync_copy(rows0, acc_sh.at[pl.ds(sid * rows_per_tile + j * K, K)])
        return c

    lax.fori_loop(0, rows_per_tile // K, zacc, 0)

    for b in range(NBUF - 1):
        pltpu.async_copy(zs_half.at[rowb.at[b]], rows[b], sg[b])
    plsc.subcore_barrier()

    def scale_chunk(rbuf, j):
        ew_row = ewb.at[j]

        @plsc.parallel_loop(0, K, unroll=unroll)
        def _(k):
            spl = plsc.load_gather(ew_row, [jnp.full((16,), k, I32)])
            for g in range(DH // 16):
                seg = rbuf[k, pl.ds(g * 16, 16)]
                rbuf[k, pl.ds(g * 16, 16)] = seg * spl

    def outer(jo, c):
        for b in range(NBUF):
            j = jo * NBUF + b
            pltpu.make_async_copy(zs_half.at[rowb.at[j]], rows[b], sg[b]).wait()
            scale_chunk(rows[b], j)
            pltpu.async_copy(rows[b], acc_sh.at[colb.at[j]], ss[b], add=True)
            bg = (b + NBUF - 1) % NBUF

            @pl.when(jnp.logical_and(j >= 1, j + NBUF - 1 < nchunks))
            def _():
                pltpu.make_async_copy(
                    rows[bg], acc_sh.at[colb.at[j - 1]], ss[bg]).wait()
                pltpu.async_copy(
                    zs_half.at[rowb.at[j + NBUF - 1]], rows[bg], sg[bg])

            @pl.when(jnp.logical_and(j == 0, j + NBUF - 1 < nchunks))
            def _():
                pltpu.async_copy(
                    zs_half.at[rowb.at[j + NBUF - 1]], rows[bg], sg[bg])
        return c

    lax.fori_loop(0, nchunks // NBUF, outer, 0)

    for b in range(NBUF):
        j = nchunks - NBUF + b
        pltpu.make_async_copy(rows[b], acc_sh.at[colb.at[j]], ss[b]).wait()
    plsc.subcore_barrier()

    def wout(j, c):
        r = sid * rows_per_tile + j * K
        pltpu.sync_copy(acc_sh.at[pl.ds(r, K)], rows0)
        pltpu.sync_copy(rows0, out_hbm.at[cid, pl.ds(r, K)])
        return c

    lax.fori_loop(0, rows_per_tile // K, wout, 0)


def _sc_message(zs, row3, col3, ew3, nchunks, unroll=2):
    kfn = functools.partial(
        pl.kernel,
        out_type=jax.ShapeDtypeStruct((NC, NPAD, DH), F32),
        mesh=_sc_mesh(),
        compiler_params=pltpu.CompilerParams(
            needs_layout_passes=False, use_tc_tiling_on_sc=False),
        scratch_types=[
            pltpu.VMEM_SHARED((NPAD, DH), F32),
            pltpu.VMEM((nchunks, K), I32),
            pltpu.VMEM((nchunks, K), I32),
            pltpu.VMEM((nchunks, K), F32),
        ] + [pltpu.VMEM((K, DH), F32)] * NBUF
          + [pltpu.SemaphoreType.DMA] * (2 * NBUF),
    )(functools.partial(_msg_body, nchunks, unroll))
    return kfn(zs, row3, col3, ew3)


def _mm_body(x_ref, w_ref, o_ref):
    o_ref[...] = jnp.dot(x_ref[...], w_ref[...], preferred_element_type=F32)


def _tc_matmul(x, w):
    return pl.pallas_call(
        _mm_body,
        grid=(NPAD // ROWBLK,),
        in_specs=[
            pl.BlockSpec((ROWBLK, D), lambda i: (i, 0)),
            pl.BlockSpec((D, D), lambda i: (0, 0)),
        ],
        out_specs=pl.BlockSpec((ROWBLK, D), lambda i: (i, 0)),
        out_shape=jax.ShapeDtypeStruct((NPAD, D), F32),
    )(x, w)


def _scale1_body(z_ref, deg_ref, zs_ref, dis_ref):
    dis = lax.rsqrt(deg_ref[...] + 1.0)
    dis_ref[...] = dis
    zs = z_ref[...] * dis
    zs_ref[0] = zs[:, :DH]
    zs_ref[1] = zs[:, DH:]


def _tc_scale1(z, deg_col):
    return pl.pallas_call(
        _scale1_body,
        grid=(NPAD // ROWBLK,),
        in_specs=[
            pl.BlockSpec((ROWBLK, D), lambda i: (i, 0)),
            pl.BlockSpec((ROWBLK, 1), lambda i: (i, 0)),
        ],
        out_specs=[
            pl.BlockSpec((NC, ROWBLK, DH), lambda i: (0, i, 0)),
            pl.BlockSpec((ROWBLK, 1), lambda i: (i, 0)),
        ],
        out_shape=[
            jax.ShapeDtypeStruct((NC, NPAD, DH), F32),
            jax.ShapeDtypeStruct((NPAD, 1), F32),
        ],
    )(z, deg_col)


def _mid_body(acc_ref, zs_ref, dis_ref, w_ref, b_ref, o_ref):
    dis = dis_ref[...]
    full = jnp.concatenate(
        [acc_ref[0] + zs_ref[0], acc_ref[1] + zs_ref[1]], axis=1)
    pre = full * dis + b_ref[...]
    h = jnp.maximum(pre, 0.0)
    z2 = jnp.dot(h, w_ref[...], preferred_element_type=F32) * dis
    o_ref[0] = z2[:, :DH]
    o_ref[1] = z2[:, DH:]


def _tc_mid(acc, zs, dis_col, w2, b1):
    return pl.pallas_call(
        _mid_body,
        grid=(NPAD // ROWBLK,),
        in_specs=[
            pl.BlockSpec((NC, ROWBLK, DH), lambda i: (0, i, 0)),
            pl.BlockSpec((NC, ROWBLK, DH), lambda i: (0, i, 0)),
            pl.BlockSpec((ROWBLK, 1), lambda i: (i, 0)),
            pl.BlockSpec((D, D), lambda i: (0, 0)),
            pl.BlockSpec((1, D), lambda i: (0, 0)),
        ],
        out_specs=pl.BlockSpec((NC, ROWBLK, DH), lambda i: (0, i, 0)),
        out_shape=jax.ShapeDtypeStruct((NC, NPAD, DH), F32),
    )(acc, zs, dis_col, w2, b1)


def _final_body(acc_ref, zs_ref, dis_ref, b_ref, o_ref):
    full = jnp.concatenate(
        [acc_ref[0] + zs_ref[0], acc_ref[1] + zs_ref[1]], axis=1)
    o_ref[...] = full * dis_ref[...] + b_ref[...]


def _tc_final(acc, zs, dis_col, b2):
    return pl.pallas_call(
        _final_body,
        grid=(NPAD // ROWBLK,),
        in_specs=[
            pl.BlockSpec((NC, ROWBLK, DH), lambda i: (0, i, 0)),
            pl.BlockSpec((NC, ROWBLK, DH), lambda i: (0, i, 0)),
            pl.BlockSpec((ROWBLK, 1), lambda i: (i, 0)),
            pl.BlockSpec((1, D), lambda i: (0, 0)),
        ],
        out_specs=pl.BlockSpec((ROWBLK, D), lambda i: (i, 0)),
        out_shape=jax.ShapeDtypeStruct((NPAD, D), F32),
    )(acc, zs, dis_col, b2)


def kernel(x, edge_index, weight, W1, b1, W2, b2):
    e = weight.shape[0]
    row = edge_index[0].astype(I32)
    col = edge_index[1].astype(I32)
    ew = weight.astype(F32)

    epw_d = ((e + NW * 16 - 1) // (NW * 16)) * 16
    e_pad_d = epw_d * NW
    col_d = jnp.pad(col, (0, e_pad_d - e))
    ew_d = jnp.pad(ew, (0, e_pad_d - e))

    nchunks = ((e + NS * K - 1) // (NS * K) + NBUF - 1) // NBUF * NBUF
    e_pad_m = NS * nchunks * K
    row3 = jnp.pad(row, (0, e_pad_m - e)).reshape(NS, nchunks, K)
    col3 = jnp.pad(col, (0, e_pad_m - e)).reshape(NS, nchunks, K)
    ew3 = jnp.pad(ew, (0, e_pad_m - e)).reshape(NS, nchunks, K)
    x_pad = jnp.pad(x, ((0, NPAD - x.shape[0]), (0, 0)))

    z1 = _tc_matmul(x_pad, W1)
    degp = _sc_degree(col_d, ew_d, epw_d)
    deg_col = jnp.sum(degp, axis=0).reshape(NPAD, 1)
    zs1, dis_col = _tc_scale1(z1, deg_col)
    acc1 = _sc_message(zs1, row3, col3, ew3, nchunks)
    zs2 = _tc_mid(acc1, zs1, dis_col, W2, b1.reshape(1, D))
    acc2 = _sc_message(zs2, row3, col3, ew3, nchunks)
    out = _tc_final(acc2, zs2, dis_col, b2.reshape(1, D))
    return out[:N_NODES]
